# baseline (speedup 1.0000x reference)
import functools

import jax
import jax.numpy as jnp
from jax import lax
from jax.experimental import pallas as pl
from jax.experimental.pallas import tpu as pltpu
from jax.experimental.pallas import tpu_sc as plsc

BATCH = 1024
SEQ = 512
NTOK = BATCH * SEQ
DIM = 128
VOCAB = 33
PAD_ID = 0
MASK_ID = 1
MLM_PROB = 0.15

NCORES = 2
NSUB = 16
NLANE = 16
NWORK = NCORES * NSUB
PER_W = NTOK // NWORK
CHUNK = 128
STEPS = PER_W // CHUNK


def _mlm_aux():
    key = jax.random.key(1)
    k1, k2, k3 = jax.random.split(key, 3)
    sel_raw = jax.random.uniform(k1, (BATCH, SEQ)) < MLM_PROB
    probs = jax.random.uniform(k2, (BATCH, SEQ))
    rtok = jax.random.randint(k3, (BATCH, SEQ), 0, VOCAB)
    rtok = jnp.where(rtok == PAD_ID, MASK_ID, rtok)
    aux = jnp.where(
        ~sel_raw, -1,
        jnp.where(probs < 0.8, MASK_ID, jnp.where(probs < 0.9, rtok, -2)))
    return aux.astype(jnp.int32).reshape(NTOK)


@functools.lru_cache(maxsize=None)
def _build_sc_embed():
    @functools.partial(
        pl.kernel,
        out_type=(
            jax.ShapeDtypeStruct((NTOK, DIM), jnp.float32),
            jax.ShapeDtypeStruct((NTOK,), jnp.int32),
        ),
        mesh=plsc.VectorSubcoreMesh(core_axis_name="c", subcore_axis_name="s"),
        scratch_types=[
            pltpu.VMEM((CHUNK,), jnp.int32),
            pltpu.VMEM((CHUNK,), jnp.int32),
            pltpu.VMEM((CHUNK,), jnp.int32),
            pltpu.VMEM((CHUNK,), jnp.int32),
            pltpu.VMEM((CHUNK, DIM), jnp.float32),
            pltpu.SemaphoreType.DMA,
        ],
    )
    def _sc_embed(x_hbm, aux_hbm, table_hbm, emb_hbm, lab_hbm,
                  ids_v, aux_v, mid_v, lab_v, rows_v, sem):
        wid = lax.axis_index("s") * NCORES + lax.axis_index("c")
        base_w = wid * PER_W

        def step(j, carry):
            base = base_w + j * CHUNK
            pltpu.sync_copy(x_hbm.at[pl.ds(base, CHUNK)], ids_v)
            pltpu.sync_copy(aux_hbm.at[pl.ds(base, CHUNK)], aux_v)
            for i in range(CHUNK // NLANE):
                sl = pl.ds(i * NLANE, NLANE)
                xv = ids_v[sl]
                av = aux_v[sl]
                sel = (xv != PAD_ID) & (av != -1)
                mid_v[sl] = jnp.where(sel & (av >= 0), av, xv)
                lab_v[sl] = jnp.where(sel, xv, -100)
            pltpu.async_copy(table_hbm.at[mid_v], rows_v, sem).wait()
            pltpu.sync_copy(rows_v, emb_hbm.at[pl.ds(base, CHUNK)])
            pltpu.sync_copy(lab_v, lab_hbm.at[pl.ds(base, CHUNK)])
            return carry

        lax.fori_loop(0, STEPS, step, 0)

    return _sc_embed


def kernel(x, table):
    aux = _mlm_aux()
    xf = x.reshape(NTOK).astype(jnp.int32)
    emb, lab = _build_sc_embed()(xf, aux, table)
    return emb.reshape(BATCH, SEQ, DIM), lab.reshape(BATCH, SEQ)


# speedup vs baseline: 1.0061x; 1.0061x over previous
import functools

import jax
import jax.numpy as jnp
from jax import lax
from jax.experimental import pallas as pl
from jax.experimental.pallas import tpu as pltpu
from jax.experimental.pallas import tpu_sc as plsc

BATCH = 1024
SEQ = 512
NTOK = BATCH * SEQ
DIM = 128
VOCAB = 33
PAD_ID = 0
MASK_ID = 1
MLM_PROB = 0.15

NCORES = 2
NSUB = 16
NLANE = 16
NWORK = NCORES * NSUB
PER_W = NTOK // NWORK
CHUNK = 128
STEPS = PER_W // CHUNK
NBUF = 4


def _mlm_aux():
    key = jax.random.key(1)
    k1, k2, k3 = jax.random.split(key, 3)
    sel_raw = jax.random.uniform(k1, (BATCH, SEQ)) < MLM_PROB
    probs = jax.random.uniform(k2, (BATCH, SEQ))
    rtok = jax.random.randint(k3, (BATCH, SEQ), 0, VOCAB)
    rtok = jnp.where(rtok == PAD_ID, MASK_ID, rtok)
    aux = jnp.where(
        ~sel_raw, -1,
        jnp.where(probs < 0.8, MASK_ID, jnp.where(probs < 0.9, rtok, -2)))
    return aux.astype(jnp.int32).reshape(NTOK)


@functools.lru_cache(maxsize=None)
def _build_sc_embed():
    @functools.partial(
        pl.kernel,
        out_type=(
            jax.ShapeDtypeStruct((NTOK, DIM), jnp.float32),
            jax.ShapeDtypeStruct((NTOK,), jnp.int32),
        ),
        mesh=plsc.VectorSubcoreMesh(core_axis_name="c", subcore_axis_name="s"),
        scratch_types=[
            pltpu.VMEM((NBUF, CHUNK), jnp.int32),
            pltpu.VMEM((NBUF, CHUNK), jnp.int32),
            pltpu.VMEM((NBUF, CHUNK), jnp.int32),
            pltpu.VMEM((NBUF, CHUNK), jnp.int32),
            pltpu.VMEM((NBUF, CHUNK, DIM), jnp.float32),
            [pltpu.SemaphoreType.DMA] * NBUF,
            [pltpu.SemaphoreType.DMA] * NBUF,
            [pltpu.SemaphoreType.DMA] * NBUF,
        ],
    )
    def _sc_embed(x_hbm, aux_hbm, table_hbm, emb_hbm, lab_hbm,
                  ids_v, aux_v, mid_v, lab_v, rows_v, gsem, wsem, isem):
        wid = lax.axis_index("s") * NCORES + lax.axis_index("c")
        base_w = wid * PER_W

        def fire_loads(s, b):
            base = base_w + s * CHUNK
            pltpu.async_copy(x_hbm.at[pl.ds(base, CHUNK)], ids_v.at[b], isem[b])
            pltpu.async_copy(aux_hbm.at[pl.ds(base, CHUNK)], aux_v.at[b], isem[b])

        def wait_loads(b):
            pltpu.make_async_copy(x_hbm.at[pl.ds(0, CHUNK)], ids_v.at[b], isem[b]).wait()
            pltpu.make_async_copy(aux_hbm.at[pl.ds(0, CHUNK)], aux_v.at[b], isem[b]).wait()

        def compute(b):
            for i in range(CHUNK // NLANE):
                sl = pl.ds(i * NLANE, NLANE)
                xv = ids_v[b, sl]
                av = aux_v[b, sl]
                sel = (xv != PAD_ID) & (av != -1)
                mid_v[b, sl] = jnp.where(sel & (av >= 0), av, xv)
                lab_v[b, sl] = jnp.where(sel, xv, -100)

        def fire_gather(b):
            pltpu.async_copy(table_hbm.at[mid_v.at[b]], rows_v.at[b], gsem[b])

        def wait_gather(b):
            pltpu.make_async_copy(
                table_hbm.at[mid_v.at[b]], rows_v.at[b], gsem[b]).wait()

        def fire_writes(s, b):
            base = base_w + s * CHUNK
            pltpu.async_copy(rows_v.at[b], emb_hbm.at[pl.ds(base, CHUNK)], wsem[b])
            pltpu.async_copy(lab_v.at[b], lab_hbm.at[pl.ds(base, CHUNK)], wsem[b])

        def wait_writes(b):
            pltpu.make_async_copy(rows_v.at[b], emb_hbm.at[pl.ds(0, CHUNK)], wsem[b]).wait()
            pltpu.make_async_copy(lab_v.at[b], lab_hbm.at[pl.ds(0, CHUNK)], wsem[b]).wait()

        for b in range(NBUF):
            fire_loads(b, b)
        for s in range(NBUF):
            b = s % NBUF
            if s >= 1:
                bp = (b + NBUF - 1) % NBUF
                wait_gather(bp)
                fire_writes(s - 1, bp)
            wait_loads(b)
            compute(b)
            fire_gather(b)
            fire_loads(s + NBUF, b)

        def group(g, carry):
            for b in range(NBUF):
                s = g * NBUF + b
                bp = (b + NBUF - 1) % NBUF
                wait_gather(bp)
                fire_writes(s - 1, bp)
                wait_writes(b)
                wait_loads(b)
                compute(b)
                fire_gather(b)
                fire_loads(s + NBUF, b)
            return carry

        lax.fori_loop(1, STEPS // NBUF - 1, group, 0)

        for s in range(STEPS - NBUF, STEPS):
            b = s % NBUF
            bp = (b + NBUF - 1) % NBUF
            wait_gather(bp)
            fire_writes(s - 1, bp)
            wait_writes(b)
            wait_loads(b)
            compute(b)
            fire_gather(b)
        bl = (STEPS - 1) % NBUF
        wait_gather(bl)
        fire_writes(STEPS - 1, bl)
        for b in range(NBUF):
            wait_writes(b)

    return _sc_embed


def kernel(x, table):
    aux = _mlm_aux()
    xf = x.reshape(NTOK).astype(jnp.int32)
    emb, lab = _build_sc_embed()(xf, aux, table)
    return emb.reshape(BATCH, SEQ, DIM), lab.reshape(BATCH, SEQ)


# speedup vs baseline: 5.3929x; 5.3602x over previous
import functools

import jax
import jax.numpy as jnp
from jax import lax
from jax.experimental import pallas as pl
from jax.experimental.pallas import tpu as pltpu
from jax.experimental.pallas import tpu_sc as plsc

BATCH = 1024
SEQ = 512
NTOK = BATCH * SEQ
DIM = 128
VOCAB = 33
PAD_ID = 0
MASK_ID = 1
MLM_PROB = 0.15

NCORES = 2
NSUB = 16
NLANE = 16
NWORK = NCORES * NSUB
PER_W = NTOK // NWORK
CHUNK = 128
STEPS = PER_W // CHUNK
NBUF = 2
GROUPS = CHUNK // NLANE
JUNROLL = 4


def _mlm_aux():
    key = jax.random.key(1)
    k1, k2, k3 = jax.random.split(key, 3)
    sel_raw = jax.random.uniform(k1, (BATCH, SEQ)) < MLM_PROB
    probs = jax.random.uniform(k2, (BATCH, SEQ))
    rtok = jax.random.randint(k3, (BATCH, SEQ), 0, VOCAB)
    rtok = jnp.where(rtok == PAD_ID, MASK_ID, rtok)
    aux = jnp.where(
        ~sel_raw, -1,
        jnp.where(probs < 0.8, MASK_ID, jnp.where(probs < 0.9, rtok, -2)))
    return aux.astype(jnp.int32).reshape(NTOK)


@functools.lru_cache(maxsize=None)
def _build_sc_embed():
    @functools.partial(
        pl.kernel,
        out_type=(
            jax.ShapeDtypeStruct((NTOK, DIM), jnp.float32),
            jax.ShapeDtypeStruct((NTOK,), jnp.int32),
        ),
        mesh=plsc.VectorSubcoreMesh(core_axis_name="c", subcore_axis_name="s"),
        compiler_params=pltpu.CompilerParams(needs_layout_passes=False),
        scratch_types=[
            pltpu.VMEM((VOCAB * DIM,), jnp.float32),
            pltpu.VMEM((NBUF, CHUNK), jnp.int32),
            pltpu.VMEM((NBUF, CHUNK), jnp.int32),
            pltpu.VMEM((CHUNK,), jnp.int32),
            pltpu.VMEM((NBUF, CHUNK), jnp.int32),
            pltpu.VMEM((NBUF, CHUNK, DIM), jnp.float32),
            [pltpu.SemaphoreType.DMA] * NBUF,
            [pltpu.SemaphoreType.DMA] * NBUF,
        ],
    )
    def _sc_embed(x_hbm, aux_hbm, table_hbm, emb_hbm, lab_hbm,
                  tab_v, ids_v, aux_v, rb_v, lab_v, rows_v, wsem, isem):
        wid = lax.axis_index("s") * NCORES + lax.axis_index("c")
        base_w = wid * PER_W

        def fire_loads(s, b):
            base = base_w + s * CHUNK
            pltpu.async_copy(x_hbm.at[pl.ds(base, CHUNK)], ids_v.at[b], isem[b])
            pltpu.async_copy(aux_hbm.at[pl.ds(base, CHUNK)], aux_v.at[b], isem[b])

        def wait_loads(b):
            pltpu.make_async_copy(x_hbm.at[pl.ds(0, CHUNK)], ids_v.at[b], isem[b]).wait()
            pltpu.make_async_copy(aux_hbm.at[pl.ds(0, CHUNK)], aux_v.at[b], isem[b]).wait()

        def fire_writes(s, b):
            base = base_w + s * CHUNK
            pltpu.async_copy(rows_v.at[b], emb_hbm.at[pl.ds(base, CHUNK)], wsem[b])
            pltpu.async_copy(lab_v.at[b], lab_hbm.at[pl.ds(base, CHUNK)], wsem[b])

        def wait_writes(b):
            pltpu.make_async_copy(rows_v.at[b], emb_hbm.at[pl.ds(0, CHUNK)], wsem[b]).wait()
            pltpu.make_async_copy(lab_v.at[b], lab_hbm.at[pl.ds(0, CHUNK)], wsem[b]).wait()

        iota = lax.iota(jnp.int32, NLANE)
        coff = [iota + c * NLANE for c in range(DIM // NLANE)]

        def compute(b):
            for i in range(GROUPS):
                sl = pl.ds(i * NLANE, NLANE)
                xv = ids_v[b, sl]
                av = aux_v[b, sl]
                sel = (xv != PAD_ID) & (av != -1)
                mid = jnp.where(sel & (av >= 0), av, xv)
                rb_v[sl] = mid * DIM
                lab_v[b, sl] = jnp.where(sel, xv, -100)

        def expand(b):
            def token(t):
                tv = jnp.full((NLANE,), t, jnp.int32)
                rj = plsc.load_gather(rb_v, [tv])
                for c in range(DIM // NLANE):
                    val = plsc.load_gather(tab_v, [rj + coff[c]])
                    rows_v[b, t, pl.ds(c * NLANE, NLANE)] = val

            def tbody(to, carry):
                for ti in range(JUNROLL):
                    token(to * JUNROLL + ti)
                return carry

            lax.fori_loop(0, CHUNK // JUNROLL, tbody, 0)

        def step(s, b, first, last):
            wait_loads(b)
            if not first:
                wait_writes(b)
            compute(b)
            expand(b)
            fire_writes(s, b)
            if not last:
                fire_loads(s + NBUF, b)

        pltpu.sync_copy(table_hbm, tab_v)
        for b in range(NBUF):
            fire_loads(b, b)
        for s in range(NBUF):
            step(s, s % NBUF, first=True, last=False)

        def group(g, carry):
            for b in range(NBUF):
                step(g * NBUF + b, b, first=False, last=False)
            return carry

        lax.fori_loop(1, STEPS // NBUF - 1, group, 0)

        for s in range(STEPS - NBUF, STEPS):
            step(s, s % NBUF, first=False, last=True)
        for b in range(NBUF):
            wait_writes(b)

    return _sc_embed


def kernel(x, table):
    aux = _mlm_aux()
    xf = x.reshape(NTOK).astype(jnp.int32)
    emb, lab = _build_sc_embed()(xf, aux, table.reshape(VOCAB * DIM))
    return emb.reshape(BATCH, SEQ, DIM), lab.reshape(BATCH, SEQ)


# speedup vs baseline: 7.0789x; 1.3127x over previous
import functools

import jax
import jax.numpy as jnp
from jax import lax
from jax.experimental import pallas as pl
from jax.experimental.pallas import tpu as pltpu
from jax.experimental.pallas import tpu_sc as plsc

BATCH = 1024
SEQ = 512
NTOK = BATCH * SEQ
DIM = 128
VOCAB = 33
PAD_ID = 0
MASK_ID = 1
MLM_PROB = 0.15

NCORES = 2
NSUB = 16
NLANE = 16
NWORK = NCORES * NSUB
PER_W = NTOK // NWORK
CHUNK = 128
STEPS = PER_W // CHUNK
NBUF = 2
GROUPS = CHUNK // NLANE


def _mlm_aux():
    key = jax.random.key(1)
    k1, k2, k3 = jax.random.split(key, 3)
    sel_raw = jax.random.uniform(k1, (BATCH, SEQ)) < MLM_PROB
    probs = jax.random.uniform(k2, (BATCH, SEQ))
    rtok = jax.random.randint(k3, (BATCH, SEQ), 0, VOCAB)
    rtok = jnp.where(rtok == PAD_ID, MASK_ID, rtok)
    aux = jnp.where(
        ~sel_raw, -1,
        jnp.where(probs < 0.8, MASK_ID, jnp.where(probs < 0.9, rtok, -2)))
    return aux.astype(jnp.int32).reshape(NTOK)


@functools.lru_cache(maxsize=None)
def _build_sc_embed():
    @functools.partial(
        pl.kernel,
        out_type=(
            jax.ShapeDtypeStruct((NTOK, DIM), jnp.float32),
            jax.ShapeDtypeStruct((NTOK,), jnp.int32),
        ),
        mesh=plsc.VectorSubcoreMesh(core_axis_name="c", subcore_axis_name="s"),
        compiler_params=pltpu.CompilerParams(needs_layout_passes=False),
        scratch_types=[
            pltpu.VMEM((VOCAB * DIM,), jnp.float32),
            pltpu.VMEM((NBUF, CHUNK), jnp.int32),
            pltpu.VMEM((NBUF, CHUNK), jnp.int32),
            pltpu.VMEM((CHUNK,), jnp.int32),
            pltpu.VMEM((NBUF, CHUNK), jnp.int32),
            pltpu.VMEM((NBUF, CHUNK, DIM), jnp.float32),
            [pltpu.SemaphoreType.DMA] * NBUF,
            [pltpu.SemaphoreType.DMA] * NBUF,
        ],
    )
    def _sc_embed(x_hbm, aux_hbm, table_hbm, emb_hbm, lab_hbm,
                  tab_v, ids_v, aux_v, rb_v, lab_v, rows_v, wsem, isem):
        wid = lax.axis_index("s") * NCORES + lax.axis_index("c")
        base_w = wid * PER_W

        def fire_loads(s, b):
            base = base_w + s * CHUNK
            pltpu.async_copy(x_hbm.at[pl.ds(base, CHUNK)], ids_v.at[b], isem[b])
            pltpu.async_copy(aux_hbm.at[pl.ds(base, CHUNK)], aux_v.at[b], isem[b])

        def wait_loads(b):
            pltpu.make_async_copy(x_hbm.at[pl.ds(0, CHUNK)], ids_v.at[b], isem[b]).wait()
            pltpu.make_async_copy(aux_hbm.at[pl.ds(0, CHUNK)], aux_v.at[b], isem[b]).wait()

        def fire_writes(s, b):
            base = base_w + s * CHUNK
            pltpu.async_copy(rows_v.at[b], emb_hbm.at[pl.ds(base, CHUNK)], wsem[b])
            pltpu.async_copy(lab_v.at[b], lab_hbm.at[pl.ds(base, CHUNK)], wsem[b])

        def wait_writes(b):
            pltpu.make_async_copy(rows_v.at[b], emb_hbm.at[pl.ds(0, CHUNK)], wsem[b]).wait()
            pltpu.make_async_copy(lab_v.at[b], lab_hbm.at[pl.ds(0, CHUNK)], wsem[b]).wait()

        iota = lax.iota(jnp.int32, NLANE)
        coff = [iota + c * NLANE for c in range(DIM // NLANE)]
        _gdims = lax.GatherDimensionNumbers(
            offset_dims=(), collapsed_slice_dims=(0,), start_index_map=(0,))

        def _lane_bcast(vec, j):
            jv = jnp.full((NLANE,), j, jnp.int32)
            return lax.gather(vec, jv[:, None], _gdims, (1,),
                              mode=lax.GatherScatterMode.PROMISE_IN_BOUNDS)

        def compute(b):
            for i in range(GROUPS):
                sl = pl.ds(i * NLANE, NLANE)
                xv = ids_v[b, sl]
                av = aux_v[b, sl]
                sel = (xv != PAD_ID) & (av != -1)
                mid = jnp.where(sel & (av >= 0), av, xv)
                rb_v[sl] = mid * DIM
                lab_v[b, sl] = jnp.where(sel, xv, -100)

        def expand(b):
            def gbody(gi, carry):
                rbg = plsc.load_gather(rb_v, [gi * NLANE + iota])
                for j in range(NLANE):
                    rj = _lane_bcast(rbg, j)
                    t = gi * NLANE + j
                    for c in range(DIM // NLANE):
                        val = plsc.load_gather(tab_v, [rj + coff[c]])
                        rows_v[b, t, pl.ds(c * NLANE, NLANE)] = val
                return carry

            lax.fori_loop(0, GROUPS, gbody, 0)

        def step(s, b, first, last):
            wait_loads(b)
            if not first:
                wait_writes(b)
            compute(b)
            expand(b)
            fire_writes(s, b)
            if not last:
                fire_loads(s + NBUF, b)

        pltpu.sync_copy(table_hbm, tab_v)
        for b in range(NBUF):
            fire_loads(b, b)
        for s in range(NBUF):
            step(s, s % NBUF, first=True, last=False)

        def group(g, carry):
            for b in range(NBUF):
                step(g * NBUF + b, b, first=False, last=False)
            return carry

        lax.fori_loop(1, STEPS // NBUF - 1, group, 0)

        for s in range(STEPS - NBUF, STEPS):
            step(s, s % NBUF, first=False, last=True)
        for b in range(NBUF):
            wait_writes(b)

    return _sc_embed


def kernel(x, table):
    with jax.ensure_compile_time_eval():
        aux = _mlm_aux()
    xf = x.reshape(NTOK).astype(jnp.int32)
    emb, lab = _build_sc_embed()(xf, aux, table.reshape(VOCAB * DIM))
    return emb.reshape(BATCH, SEQ, DIM), lab.reshape(BATCH, SEQ)


# speedup vs baseline: 26.2504x; 3.7082x over previous
import functools

import jax
import jax.numpy as jnp
from jax import lax
from jax.experimental import pallas as pl
from jax.experimental.pallas import tpu as pltpu
from jax.experimental.pallas import tpu_sc as plsc

BATCH = 1024
SEQ = 512
NTOK = BATCH * SEQ
DIM = 128
VOCAB = 33
PAD_ID = 0
MASK_ID = 1
MLM_PROB = 0.15

NCORES = 2
NSUB = 16
NLANE = 16
NWORK = NCORES * NSUB
PER_W = NTOK // NWORK
CHUNK = 128
STEPS = PER_W // CHUNK
NBUF = 2
GROUPS = CHUNK // NLANE


def _mlm_aux():
    key = jax.random.key(1)
    k1, k2, k3 = jax.random.split(key, 3)
    sel_raw = jax.random.uniform(k1, (BATCH, SEQ)) < MLM_PROB
    probs = jax.random.uniform(k2, (BATCH, SEQ))
    rtok = jax.random.randint(k3, (BATCH, SEQ), 0, VOCAB)
    rtok = jnp.where(rtok == PAD_ID, MASK_ID, rtok)
    aux = jnp.where(
        ~sel_raw, -1,
        jnp.where(probs < 0.8, MASK_ID, jnp.where(probs < 0.9, rtok, -2)))
    return aux.astype(jnp.int32).reshape(NTOK)


@functools.lru_cache(maxsize=None)
def _build_sc_embed():
    @functools.partial(
        pl.kernel,
        out_type=(
            jax.ShapeDtypeStruct((NTOK, DIM), jnp.float32),
            jax.ShapeDtypeStruct((NTOK,), jnp.int32),
        ),
        mesh=plsc.VectorSubcoreMesh(core_axis_name="c", subcore_axis_name="s"),
        compiler_params=pltpu.CompilerParams(needs_layout_passes=False),
        scratch_types=[
            pltpu.VMEM((VOCAB * DIM,), jnp.float32),
            pltpu.VMEM((NBUF, CHUNK), jnp.int32),
            pltpu.VMEM((NBUF, CHUNK), jnp.int32),
            pltpu.VMEM((CHUNK,), jnp.int32),
            pltpu.VMEM((NBUF, CHUNK), jnp.int32),
            pltpu.VMEM((NBUF, CHUNK, DIM), jnp.float32),
            [pltpu.SemaphoreType.DMA] * NBUF,
            [pltpu.SemaphoreType.DMA] * NBUF,
        ],
    )
    def _sc_embed(x_hbm, aux_hbm, table_hbm, emb_hbm, lab_hbm,
                  tab_v, ids_v, aux_v, rb_v, lab_v, rows_v, wsem, isem):
        wid = lax.axis_index("s") * NCORES + lax.axis_index("c")
        base_w = wid * PER_W

        def fire_loads(s, b):
            base = base_w + s * CHUNK
            pltpu.async_copy(x_hbm.at[pl.ds(base, CHUNK)], ids_v.at[b], isem[b])
            pltpu.async_copy(aux_hbm.at[pl.ds(base, CHUNK)], aux_v.at[b], isem[b])

        def wait_loads(b):
            pltpu.make_async_copy(x_hbm.at[pl.ds(0, CHUNK)], ids_v.at[b], isem[b]).wait()
            pltpu.make_async_copy(aux_hbm.at[pl.ds(0, CHUNK)], aux_v.at[b], isem[b]).wait()

        def fire_writes(s, b):
            base = base_w + s * CHUNK
            pltpu.async_copy(rows_v.at[b], emb_hbm.at[pl.ds(base, CHUNK)], wsem[b])
            pltpu.async_copy(lab_v.at[b], lab_hbm.at[pl.ds(base, CHUNK)], wsem[b])

        def wait_writes(b):
            pltpu.make_async_copy(rows_v.at[b], emb_hbm.at[pl.ds(0, CHUNK)], wsem[b]).wait()
            pltpu.make_async_copy(lab_v.at[b], lab_hbm.at[pl.ds(0, CHUNK)], wsem[b]).wait()

        iota = lax.iota(jnp.int32, NLANE)
        coff = [iota + c * NLANE for c in range(DIM // NLANE)]
        _gdims = lax.GatherDimensionNumbers(
            offset_dims=(), collapsed_slice_dims=(0,), start_index_map=(0,))

        def _lane_bcast(vec, j):
            jv = jnp.full((NLANE,), j, jnp.int32)
            return lax.gather(vec, jv[:, None], _gdims, (1,),
                              mode=lax.GatherScatterMode.PROMISE_IN_BOUNDS)

        def compute(b):
            for i in range(GROUPS):
                sl = pl.ds(i * NLANE, NLANE)
                xv = ids_v[b, sl]
                av = aux_v[b, sl]
                sel = (xv != PAD_ID) & (av != -1)
                mid = jnp.where(sel & (av >= 0), av, xv)
                rb_v[sl] = mid * DIM
                lab_v[b, sl] = jnp.where(sel, xv, -100)

        def expand(b):
            @plsc.parallel_loop(0, CHUNK, unroll=4)
            def token(t):
                tv = jnp.full((NLANE,), t, jnp.int32)
                rj = plsc.load_gather(rb_v, [tv])
                for c in range(DIM // NLANE):
                    val = plsc.load_gather(tab_v, [rj + coff[c]])
                    rows_v[b, t, pl.ds(c * NLANE, NLANE)] = val

        def step(s, b, first, last):
            wait_loads(b)
            if not first:
                wait_writes(b)
            compute(b)
            expand(b)
            fire_writes(s, b)
            if not last:
                fire_loads(s + NBUF, b)

        pltpu.sync_copy(table_hbm, tab_v)
        for b in range(NBUF):
            fire_loads(b, b)
        for s in range(NBUF):
            step(s, s % NBUF, first=True, last=False)

        def group(g, carry):
            for b in range(NBUF):
                step(g * NBUF + b, b, first=False, last=False)
            return carry

        lax.fori_loop(1, STEPS // NBUF - 1, group, 0)

        for s in range(STEPS - NBUF, STEPS):
            step(s, s % NBUF, first=False, last=True)
        for b in range(NBUF):
            wait_writes(b)

    return _sc_embed


def kernel(x, table):
    with jax.ensure_compile_time_eval():
        aux = _mlm_aux()
    xf = x.reshape(NTOK).astype(jnp.int32)
    emb, lab = _build_sc_embed()(xf, aux, table.reshape(VOCAB * DIM))
    return emb.reshape(BATCH, SEQ, DIM), lab.reshape(BATCH, SEQ)


# speedup vs baseline: 28.7566x; 1.0955x over previous
import functools

import jax
import jax.numpy as jnp
from jax import lax
from jax.experimental import pallas as pl
from jax.experimental.pallas import tpu as pltpu
from jax.experimental.pallas import tpu_sc as plsc

BATCH = 1024
SEQ = 512
NTOK = BATCH * SEQ
DIM = 128
VOCAB = 33
PAD_ID = 0
MASK_ID = 1
MLM_PROB = 0.15

NCORES = 2
NSUB = 16
NLANE = 16
NWORK = NCORES * NSUB
PER_W = NTOK // NWORK
CHUNK = 256
STEPS = PER_W // CHUNK
NBUF = 2
GROUPS = CHUNK // NLANE


def _mlm_aux():
    key = jax.random.key(1)
    k1, k2, k3 = jax.random.split(key, 3)
    sel_raw = jax.random.uniform(k1, (BATCH, SEQ)) < MLM_PROB
    probs = jax.random.uniform(k2, (BATCH, SEQ))
    rtok = jax.random.randint(k3, (BATCH, SEQ), 0, VOCAB)
    rtok = jnp.where(rtok == PAD_ID, MASK_ID, rtok)
    aux = jnp.where(
        ~sel_raw, -1,
        jnp.where(probs < 0.8, MASK_ID, jnp.where(probs < 0.9, rtok, -2)))
    return aux.astype(jnp.int32).reshape(NTOK)


@functools.lru_cache(maxsize=None)
def _build_sc_embed():
    @functools.partial(
        pl.kernel,
        out_type=(
            jax.ShapeDtypeStruct((NTOK, DIM), jnp.float32),
            jax.ShapeDtypeStruct((NTOK,), jnp.int32),
        ),
        mesh=plsc.VectorSubcoreMesh(core_axis_name="c", subcore_axis_name="s"),
        compiler_params=pltpu.CompilerParams(needs_layout_passes=False),
        scratch_types=[
            pltpu.VMEM((VOCAB * DIM,), jnp.float32),
            pltpu.VMEM((NBUF, CHUNK), jnp.int32),
            pltpu.VMEM((NBUF, CHUNK), jnp.int32),
            pltpu.VMEM((CHUNK,), jnp.int32),
            pltpu.VMEM((NBUF, CHUNK), jnp.int32),
            pltpu.VMEM((NBUF, CHUNK, DIM), jnp.float32),
            [pltpu.SemaphoreType.DMA] * NBUF,
            [pltpu.SemaphoreType.DMA] * NBUF,
        ],
    )
    def _sc_embed(x_hbm, aux_hbm, table_hbm, emb_hbm, lab_hbm,
                  tab_v, ids_v, aux_v, rb_v, lab_v, rows_v, wsem, isem):
        wid = lax.axis_index("s") * NCORES + lax.axis_index("c")
        base_w = wid * PER_W

        def fire_loads(s, b):
            base = base_w + s * CHUNK
            pltpu.async_copy(x_hbm.at[pl.ds(base, CHUNK)], ids_v.at[b], isem[b])
            pltpu.async_copy(aux_hbm.at[pl.ds(base, CHUNK)], aux_v.at[b], isem[b])

        def wait_loads(b):
            pltpu.make_async_copy(x_hbm.at[pl.ds(0, CHUNK)], ids_v.at[b], isem[b]).wait()
            pltpu.make_async_copy(aux_hbm.at[pl.ds(0, CHUNK)], aux_v.at[b], isem[b]).wait()

        def fire_writes(s, b):
            base = base_w + s * CHUNK
            pltpu.async_copy(rows_v.at[b], emb_hbm.at[pl.ds(base, CHUNK)], wsem[b])
            pltpu.async_copy(lab_v.at[b], lab_hbm.at[pl.ds(base, CHUNK)], wsem[b])

        def wait_writes(b):
            pltpu.make_async_copy(rows_v.at[b], emb_hbm.at[pl.ds(0, CHUNK)], wsem[b]).wait()
            pltpu.make_async_copy(lab_v.at[b], lab_hbm.at[pl.ds(0, CHUNK)], wsem[b]).wait()

        iota = lax.iota(jnp.int32, NLANE)
        coff = [iota + c * NLANE for c in range(DIM // NLANE)]
        _gdims = lax.GatherDimensionNumbers(
            offset_dims=(), collapsed_slice_dims=(0,), start_index_map=(0,))

        def _lane_bcast(vec, j):
            jv = jnp.full((NLANE,), j, jnp.int32)
            return lax.gather(vec, jv[:, None], _gdims, (1,),
                              mode=lax.GatherScatterMode.PROMISE_IN_BOUNDS)

        def compute(b):
            for i in range(GROUPS):
                sl = pl.ds(i * NLANE, NLANE)
                xv = ids_v[b, sl]
                av = aux_v[b, sl]
                sel = (xv != PAD_ID) & (av != -1)
                mid = jnp.where(sel & (av >= 0), av, xv)
                rb_v[sl] = mid * DIM
                lab_v[b, sl] = jnp.where(sel, xv, -100)

        def expand(b):
            @plsc.parallel_loop(0, CHUNK, unroll=8)
            def token(t):
                tv = jnp.full((NLANE,), t, jnp.int32)
                rj = plsc.load_gather(rb_v, [tv])
                for c in range(DIM // NLANE):
                    val = plsc.load_gather(tab_v, [rj + coff[c]])
                    rows_v[b, t, pl.ds(c * NLANE, NLANE)] = val

        def step(s, b, first, last):
            wait_loads(b)
            if not first:
                wait_writes(b)
            compute(b)
            expand(b)
            fire_writes(s, b)
            if not last:
                fire_loads(s + NBUF, b)

        pltpu.sync_copy(table_hbm, tab_v)
        for b in range(NBUF):
            fire_loads(b, b)
        for s in range(NBUF):
            step(s, s % NBUF, first=True, last=False)

        def group(g, carry):
            for b in range(NBUF):
                step(g * NBUF + b, b, first=False, last=False)
            return carry

        lax.fori_loop(1, STEPS // NBUF - 1, group, 0)

        for s in range(STEPS - NBUF, STEPS):
            step(s, s % NBUF, first=False, last=True)
        for b in range(NBUF):
            wait_writes(b)

    return _sc_embed


def kernel(x, table):
    with jax.ensure_compile_time_eval():
        aux = _mlm_aux()
    xf = x.reshape(NTOK).astype(jnp.int32)
    emb, lab = _build_sc_embed()(xf, aux, table.reshape(VOCAB * DIM))
    return emb.reshape(BATCH, SEQ, DIM), lab.reshape(BATCH, SEQ)
